# baseline (speedup 1.0000x reference)
import functools

import jax
import jax.numpy as jnp
from jax import lax
from jax.experimental import pallas as pl
from jax.experimental.pallas import tpu as pltpu
from jax.experimental.pallas import tpu_sc as plsc

N_NODES = 10000
D = 128
NC = 2
NS = 16
K = 128
HALF = 5120
TBL = 5248
TRASH = HALF
TBL_PER_TILE = TBL // NS
OUT_PER_TILE = HALF // NS
N_PAD = 2 * HALF

NV = 64
EPV = 320000 // NV
CAP = 5200
DUMP = 5136
CNTPOS = 5184
BPT = NV // NS
NGRP = 5120 // 16


def _prefix16(x):
  lanes = lax.iota(jnp.int32, 16)
  p = x
  for sh in (1, 2, 4, 8):
    idx = jnp.maximum(lanes - sh, 0)
    g = p.at[idx].get(mode="promise_in_bounds")
    p = p + jnp.where(lanes >= sh, g, 0)
  return p


def _part_body(src_hbm, dst_hbm, pkp_hbm, src_blk, dst_blk,
               pos_st, vpk_st, pos16, vpk16, bpk_sh):
  c = lax.axis_index("c")
  s = lax.axis_index("s")
  half_base = c * HALF
  lanes = lax.iota(jnp.int32, 16)
  neg16 = jnp.full((16,), -1, jnp.int32)
  trash_pk = lanes + jnp.int32(TRASH << 16)

  for j in range(BPT):
    v = BPT * s + j
    abs_base = v * CAP
    pltpu.sync_copy(src_hbm.at[pl.ds(v * EPV, EPV)],
                    src_blk.at[pl.ds(0, EPV)])
    pltpu.sync_copy(dst_hbm.at[pl.ds(v * EPV, EPV)],
                    dst_blk.at[pl.ds(0, EPV)])
    d = dst_blk[pl.ds(EPV - 8, 16)]
    dst_blk[pl.ds(EPV - 8, 16)] = jnp.where(lanes < 8, d, neg16)
    for q in range(EPV + 8, NGRP * 16, 16):
      dst_blk[pl.ds(q, 16)] = neg16

    def batch(bi, cnt):
      for gi in range(8):
        g = bi * 8 + gi
        s16 = src_blk[pl.ds(g * 16, 16)]
        d16 = dst_blk[pl.ds(g * 16, 16)]
        loc = d16 - half_base
        ok = (loc >= 0) & (loc < HALF)
        ok_i = jnp.where(ok, jnp.int32(1), jnp.int32(0))
        pref = _prefix16(ok_i)
        pos = jnp.where(ok, abs_base + cnt + pref - 1,
                        abs_base + DUMP + lanes)
        pos_st[pl.ds(gi * 16, 16)] = pos
        vpk_st[pl.ds(gi * 16, 16)] = s16 | (loc << 16)
        cnt = cnt + pref[15]
      pltpu.sync_copy(vpk_st, bpk_sh.at[pos_st])
      return cnt

    cnt = lax.fori_loop(0, NGRP // 8, batch, jnp.int32(0))

    nch = (cnt + (K - 1)) // K
    for gi in range(8):
      pos_st[pl.ds(gi * 16, 16)] = abs_base + cnt + gi * 16 + lanes
      vpk_st[pl.ds(gi * 16, 16)] = trash_pk
    pltpu.sync_copy(vpk_st, bpk_sh.at[pos_st])
    pos16[pl.ds(0, 16)] = abs_base + CNTPOS + lanes
    vpk16[pl.ds(0, 16)] = jnp.zeros((16,), jnp.int32) + nch
    pltpu.sync_copy(vpk16, bpk_sh.at[pos16])

  for j in range(BPT):
    v = BPT * s + j
    pltpu.sync_copy(bpk_sh.at[pl.ds(v * CAP, CAP)], src_blk)
    pltpu.sync_copy(src_blk, pkp_hbm.at[c, v])


def _make_part():
  mesh = plsc.VectorSubcoreMesh(core_axis_name="c", subcore_axis_name="s",
                                num_cores=NC)
  return pl.kernel(
      _part_body,
      out_type=jax.ShapeDtypeStruct((NC, NV, CAP), jnp.int32),
      mesh=mesh,
      scratch_types=[
          pltpu.VMEM((CAP,), jnp.int32),
          pltpu.VMEM((CAP,), jnp.int32),
          pltpu.VMEM((128,), jnp.int32),
          pltpu.VMEM((128,), jnp.int32),
          pltpu.VMEM((16,), jnp.int32),
          pltpu.VMEM((16,), jnp.int32),
          pltpu.VMEM_SHARED((NV * CAP,), jnp.int32),
      ],
  )


def _agg_body(h_hbm, pkp_hbm, out_hbm, pk_blk,
              srcc0, srcc1, srcc2, dstc0, dstc1, dstc2,
              rows0, rows1, rows2,
              gsem0, gsem1, gsem2, ssem0, ssem1, ssem2,
              agg_sh):
  c = lax.axis_index("c")
  s = lax.axis_index("s")
  half_base = c * HALF
  srcc = (srcc0, srcc1, srcc2)
  dstc = (dstc0, dstc1, dstc2)
  rows = (rows0, rows1, rows2)
  gsem = (gsem0, gsem1, gsem2)
  ssem = (ssem0, ssem1, ssem2)

  zeros16 = jnp.zeros((16,), jnp.float32)

  def zrow(i, _):
    for j in range(8):
      rows0[i, pl.ds(j * 16, 16)] = zeros16
    return 0

  lax.fori_loop(0, K, zrow, 0)
  nfull = TBL_PER_TILE // K
  for q in range(nfull):
    pltpu.sync_copy(rows0, agg_sh.at[pl.ds(s * TBL_PER_TILE + q * K, K)])
  rem = TBL_PER_TILE - nfull * K
  if rem:
    pltpu.sync_copy(rows0.at[pl.ds(0, rem)],
                    agg_sh.at[pl.ds(s * TBL_PER_TILE + nfull * K, rem)])
  plsc.subcore_barrier()

  def stage(g, b):
    for j in range(K // 16):
      e = pk_blk[pl.ds(g * K + j * 16, 16)]
      srcc[b][pl.ds(j * 16, 16)] = e & 0xFFFF
      dstc[b][pl.ds(j * 16, 16)] = e >> 16

  def start_gather(g, b):
    pltpu.async_copy(h_hbm.at[srcc[b]], rows[b], gsem[b])

  def wait_gather(b):
    pltpu.make_async_copy(h_hbm.at[srcc[b]], rows[b], gsem[b]).wait()

  def start_scatter(b):
    pltpu.async_copy(rows[b], agg_sh.at[dstc[b]], ssem[b], add=True)

  def wait_scatter(b):
    pltpu.make_async_copy(rows[b], agg_sh.at[dstc[b]], ssem[b]).wait()

  for j in range(BPT):
    v = BPT * s + j
    pltpu.sync_copy(pkp_hbm.at[c, v], pk_blk)
    n_ch = pk_blk[pl.ds(CNTPOS, 16)][0]

    for b in range(2):

      @pl.when(b < n_ch)
      def _():
        stage(b, b)
        start_gather(b, b)

    def triple(tt, _):
      for b in range(3):
        g = 3 * tt + b

        @pl.when(g < n_ch)
        def _():
          wait_gather(b)
          start_scatter(b)
          bb = (b + 2) % 3

          @pl.when(g >= 1)
          def _():
            wait_scatter(bb)

          @pl.when(g + 2 < n_ch)
          def _():
            stage(g + 2, bb)
            start_gather(g + 2, bb)
      return 0

    lax.fori_loop(0, (n_ch + 2) // 3, triple, 0)

    for b in range(3):

      @pl.when((n_ch >= 1) & ((n_ch - 1) % 3 == b))
      def _():
        wait_scatter(b)

  plsc.subcore_barrier()

  nf = OUT_PER_TILE // K
  for q in range(nf):
    pltpu.sync_copy(agg_sh.at[pl.ds(s * OUT_PER_TILE + q * K, K)],
                    rows[q % 3])
    pltpu.sync_copy(
        rows[q % 3],
        out_hbm.at[pl.ds(half_base + s * OUT_PER_TILE + q * K, K)])
  remw = OUT_PER_TILE - nf * K
  if remw:
    pltpu.sync_copy(agg_sh.at[pl.ds(s * OUT_PER_TILE + nf * K, remw)],
                    rows[nf % 3].at[pl.ds(0, remw)])
    pltpu.sync_copy(
        rows[nf % 3].at[pl.ds(0, remw)],
        out_hbm.at[pl.ds(half_base + s * OUT_PER_TILE + nf * K, remw)])


def _make_agg():
  mesh = plsc.VectorSubcoreMesh(core_axis_name="c", subcore_axis_name="s",
                                num_cores=NC)
  return pl.kernel(
      _agg_body,
      out_type=jax.ShapeDtypeStruct((N_PAD, D), jnp.float32),
      mesh=mesh,
      scratch_types=[
          pltpu.VMEM((CAP,), jnp.int32),
          pltpu.VMEM((K,), jnp.int32),
          pltpu.VMEM((K,), jnp.int32),
          pltpu.VMEM((K,), jnp.int32),
          pltpu.VMEM((K,), jnp.int32),
          pltpu.VMEM((K,), jnp.int32),
          pltpu.VMEM((K,), jnp.int32),
          pltpu.VMEM((K, D), jnp.float32),
          pltpu.VMEM((K, D), jnp.float32),
          pltpu.VMEM((K, D), jnp.float32),
          pltpu.SemaphoreType.DMA,
          pltpu.SemaphoreType.DMA,
          pltpu.SemaphoreType.DMA,
          pltpu.SemaphoreType.DMA,
          pltpu.SemaphoreType.DMA,
          pltpu.SemaphoreType.DMA,
          pltpu.VMEM_SHARED((TBL, D), jnp.float32),
      ],
  )


def _dense_mid_body(a_ref, h_ref, wrelT_ref, wrootT_ref, b_ref, o_ref):
  y = jnp.dot(a_ref[...], wrelT_ref[...], preferred_element_type=jnp.float32)
  y += jnp.dot(h_ref[...], wrootT_ref[...], preferred_element_type=jnp.float32)
  y += b_ref[...]
  o_ref[...] = jnp.maximum(y, 0.0)


def _final_body(h_ref, wgT_ref, bg_ref, o_ref):
  o_ref[...] = (
      jnp.dot(h_ref[...], wgT_ref[...], preferred_element_type=jnp.float32)
      + bg_ref[...])


_R = 2000


def _dense_mid(agg, h, wrelT, wrootT, b2d):
  grid = (N_NODES // _R,)
  return pl.pallas_call(
      _dense_mid_body,
      grid=grid,
      in_specs=[
          pl.BlockSpec((_R, D), lambda i: (i, 0)),
          pl.BlockSpec((_R, D), lambda i: (i, 0)),
          pl.BlockSpec((D, D), lambda i: (0, 0)),
          pl.BlockSpec((D, D), lambda i: (0, 0)),
          pl.BlockSpec((1, D), lambda i: (0, 0)),
      ],
      out_specs=pl.BlockSpec((_R, D), lambda i: (i, 0)),
      out_shape=jax.ShapeDtypeStruct((N_NODES, D), jnp.float32),
  )(agg, h, wrelT, wrootT, b2d)


def _final(h, wgT, bg2d):
  grid = (N_NODES // _R,)
  return pl.pallas_call(
      _final_body,
      grid=grid,
      in_specs=[
          pl.BlockSpec((_R, D), lambda i: (i, 0)),
          pl.BlockSpec((D, D), lambda i: (0, 0)),
          pl.BlockSpec((1, D), lambda i: (0, 0)),
      ],
      out_specs=pl.BlockSpec((_R, D), lambda i: (i, 0)),
      out_shape=jax.ShapeDtypeStruct((N_NODES, D), jnp.float32),
  )(h, wgT, bg2d)


def kernel(x, edge_index, Wrel0, brel0, Wroot0, Wrel1, brel1, Wroot1, Wrel2,
           brel2, Wroot2, Wg, bg):
  n_edges = edge_index.shape[1]
  src = edge_index[0]
  dst = edge_index[1]
  pkp = _make_part()(src, dst)
  agg_fn = _make_agg()

  wrelT = jnp.stack([Wrel0.T, Wrel1.T, Wrel2.T])
  wrootT = jnp.stack([Wroot0.T, Wroot1.T, Wroot2.T])
  b2 = jnp.stack([brel0.reshape(1, D), brel1.reshape(1, D),
                  brel2.reshape(1, D)])

  def layer(h, ws):
    wrelT_i, wrootT_i, b_i = ws
    agg = agg_fn(h, pkp)
    h2 = _dense_mid(agg, h, wrelT_i, wrootT_i, b_i)
    return h2, None

  h3, _ = lax.scan(layer, x, (wrelT, wrootT, b2))
  return _final(h3, Wg.T, bg.reshape(1, D))


# speedup vs baseline: 1.0446x; 1.0446x over previous
import functools

import jax
import jax.numpy as jnp
from jax import lax
from jax.experimental import pallas as pl
from jax.experimental.pallas import tpu as pltpu
from jax.experimental.pallas import tpu_sc as plsc

N_NODES = 10000
D = 128
NC = 2
NS = 16
K = 80
HALF = 5120
TBL = 5248
TRASH = HALF
TBL_PER_TILE = TBL // NS
OUT_PER_TILE = HALF // NS
N_PAD = 2 * HALF

NV = 64
EPV = 320000 // NV
CAP = 5200
DUMP = 5088
CNTPOS = 5184
BPT = NV // NS
NGRP = 5120 // 16


def _prefix16(x):
  lanes = lax.iota(jnp.int32, 16)
  p = x
  for sh in (1, 2, 4, 8):
    idx = jnp.maximum(lanes - sh, 0)
    g = p.at[idx].get(mode="promise_in_bounds")
    p = p + jnp.where(lanes >= sh, g, 0)
  return p


def _part_body(src_hbm, dst_hbm, pkp_hbm, src_blk, dst_blk,
               pos_st, vpk_st, bpk_sh):
  c = lax.axis_index("c")
  s = lax.axis_index("s")
  half_base = c * HALF
  lanes = lax.iota(jnp.int32, 16)
  neg16 = jnp.full((16,), -1, jnp.int32)
  trash_pk = lanes + jnp.int32(TRASH << 16)

  for j in range(BPT):
    v = BPT * s + j
    abs_base = v * CAP
    pltpu.sync_copy(src_hbm.at[pl.ds(v * EPV, EPV)],
                    src_blk.at[pl.ds(0, EPV)])
    pltpu.sync_copy(dst_hbm.at[pl.ds(v * EPV, EPV)],
                    dst_blk.at[pl.ds(0, EPV)])
    d = dst_blk[pl.ds(EPV - 8, 16)]
    dst_blk[pl.ds(EPV - 8, 16)] = jnp.where(lanes < 8, d, neg16)
    for q in range(EPV + 8, NGRP * 16, 16):
      dst_blk[pl.ds(q, 16)] = neg16

    def batch(bi, cnt):
      for gi in range(8):
        g = bi * 8 + gi
        s16 = src_blk[pl.ds(g * 16, 16)]
        d16 = dst_blk[pl.ds(g * 16, 16)]
        loc = d16 - half_base
        ok = (loc >= 0) & (loc < HALF)
        ok_i = jnp.where(ok, jnp.int32(1), jnp.int32(0))
        pref = _prefix16(ok_i)
        pos = jnp.where(ok, abs_base + cnt + pref - 1,
                        abs_base + DUMP + lanes)
        pos_st[pl.ds(gi * 16, 16)] = pos
        vpk_st[pl.ds(gi * 16, 16)] = s16 | (loc << 16)
        cnt = cnt + pref[15]
      pltpu.sync_copy(vpk_st, bpk_sh.at[pos_st])
      return cnt

    cnt = lax.fori_loop(0, NGRP // 8, batch, jnp.int32(0))

    nch = (cnt + (K - 1)) // K
    for gi in range(8):
      if gi < 5:
        pos_st[pl.ds(gi * 16, 16)] = abs_base + cnt + gi * 16 + lanes
        vpk_st[pl.ds(gi * 16, 16)] = trash_pk
      elif gi == 5:
        pos_st[pl.ds(gi * 16, 16)] = abs_base + CNTPOS + lanes
        vpk_st[pl.ds(gi * 16, 16)] = jnp.zeros((16,), jnp.int32) + nch
      else:
        pos_st[pl.ds(gi * 16, 16)] = abs_base + DUMP + lanes
        vpk_st[pl.ds(gi * 16, 16)] = trash_pk
    pltpu.sync_copy(vpk_st, bpk_sh.at[pos_st])

  for j in range(BPT):
    v = BPT * s + j
    pltpu.sync_copy(bpk_sh.at[pl.ds(v * CAP, CAP)], src_blk)
    pltpu.sync_copy(src_blk, pkp_hbm.at[c, v])


def _make_part():
  mesh = plsc.VectorSubcoreMesh(core_axis_name="c", subcore_axis_name="s",
                                num_cores=NC)
  return pl.kernel(
      _part_body,
      out_type=jax.ShapeDtypeStruct((NC, NV, CAP), jnp.int32),
      mesh=mesh,
      scratch_types=[
          pltpu.VMEM((CAP,), jnp.int32),
          pltpu.VMEM((CAP,), jnp.int32),
          pltpu.VMEM((128,), jnp.int32),
          pltpu.VMEM((128,), jnp.int32),
          pltpu.VMEM_SHARED((NV * CAP,), jnp.int32),
      ],
  )


def _agg_body(h_hbm, pkp_hbm, out_hbm, pk_blk,
              srcc0, srcc1, srcc2, srcc3, srcc4,
              dstc0, dstc1, dstc2, dstc3, dstc4,
              rows0, rows1, rows2, rows3, rows4,
              gsem0, gsem1, gsem2, gsem3, gsem4,
              ssem0, ssem1, ssem2, ssem3, ssem4,
              agg_sh):
  c = lax.axis_index("c")
  s = lax.axis_index("s")
  half_base = c * HALF
  srcc = (srcc0, srcc1, srcc2, srcc3, srcc4)
  dstc = (dstc0, dstc1, dstc2, dstc3, dstc4)
  rows = (rows0, rows1, rows2, rows3, rows4)
  gsem = (gsem0, gsem1, gsem2, gsem3, gsem4)
  ssem = (ssem0, ssem1, ssem2, ssem3, ssem4)

  zeros16 = jnp.zeros((16,), jnp.float32)

  def zrow(i, _):
    for j in range(8):
      rows0[i, pl.ds(j * 16, 16)] = zeros16
    return 0

  lax.fori_loop(0, K, zrow, 0)
  for q in range(4):
    pltpu.sync_copy(rows0, agg_sh.at[pl.ds(s * TBL_PER_TILE + q * K, K)])
  pltpu.sync_copy(rows0.at[pl.ds(0, TBL_PER_TILE - 4 * K)],
                  agg_sh.at[pl.ds(s * TBL_PER_TILE + 4 * K,
                                  TBL_PER_TILE - 4 * K)])
  plsc.subcore_barrier()

  def stage(g, b):
    for j in range(K // 16):
      e = pk_blk[pl.ds(g * K + j * 16, 16)]
      srcc[b][pl.ds(j * 16, 16)] = e & 0xFFFF
      dstc[b][pl.ds(j * 16, 16)] = e >> 16

  def start_gather(g, b):
    pltpu.async_copy(h_hbm.at[srcc[b]], rows[b], gsem[b])

  def wait_gather(b):
    pltpu.make_async_copy(h_hbm.at[srcc[b]], rows[b], gsem[b]).wait()

  def start_scatter(b):
    pltpu.async_copy(rows[b], agg_sh.at[dstc[b]], ssem[b], add=True)

  def wait_scatter(b):
    pltpu.make_async_copy(rows[b], agg_sh.at[dstc[b]], ssem[b]).wait()

  for j in range(BPT):
    v = BPT * s + j
    pltpu.sync_copy(pkp_hbm.at[c, v], pk_blk)
    n_ch = pk_blk[pl.ds(CNTPOS, 16)][0]

    for b in range(3):

      @pl.when(b < n_ch)
      def _():
        stage(b, b)
        start_gather(b, b)

    def quint(qq, _):
      for b in range(5):
        g = 5 * qq + b

        @pl.when(g < n_ch)
        def _():
          wait_gather(b)
          start_scatter(b)
          bb = (b + 3) % 5

          @pl.when(g >= 2)
          def _():
            wait_scatter(bb)

          @pl.when(g + 3 < n_ch)
          def _():
            stage(g + 3, bb)
            start_gather(g + 3, bb)
      return 0

    lax.fori_loop(0, (n_ch + 4) // 5, quint, 0)

    for b in range(5):

      @pl.when((n_ch >= 2) & ((n_ch - 2) % 5 == b))
      def _():
        wait_scatter(b)
    for b in range(5):

      @pl.when((n_ch >= 1) & ((n_ch - 1) % 5 == b))
      def _():
        wait_scatter(b)

  plsc.subcore_barrier()

  for q in range(4):
    pltpu.sync_copy(agg_sh.at[pl.ds(s * OUT_PER_TILE + q * K, K)], rows[q])
    pltpu.sync_copy(
        rows[q], out_hbm.at[pl.ds(half_base + s * OUT_PER_TILE + q * K, K)])


def _make_agg():
  mesh = plsc.VectorSubcoreMesh(core_axis_name="c", subcore_axis_name="s",
                                num_cores=NC)
  return pl.kernel(
      _agg_body,
      out_type=jax.ShapeDtypeStruct((N_PAD, D), jnp.float32),
      mesh=mesh,
      scratch_types=[
          pltpu.VMEM((CAP,), jnp.int32),
          pltpu.VMEM((K,), jnp.int32),
          pltpu.VMEM((K,), jnp.int32),
          pltpu.VMEM((K,), jnp.int32),
          pltpu.VMEM((K,), jnp.int32),
          pltpu.VMEM((K,), jnp.int32),
          pltpu.VMEM((K,), jnp.int32),
          pltpu.VMEM((K,), jnp.int32),
          pltpu.VMEM((K,), jnp.int32),
          pltpu.VMEM((K,), jnp.int32),
          pltpu.VMEM((K,), jnp.int32),
          pltpu.VMEM((K, D), jnp.float32),
          pltpu.VMEM((K, D), jnp.float32),
          pltpu.VMEM((K, D), jnp.float32),
          pltpu.VMEM((K, D), jnp.float32),
          pltpu.VMEM((K, D), jnp.float32),
          pltpu.SemaphoreType.DMA,
          pltpu.SemaphoreType.DMA,
          pltpu.SemaphoreType.DMA,
          pltpu.SemaphoreType.DMA,
          pltpu.SemaphoreType.DMA,
          pltpu.SemaphoreType.DMA,
          pltpu.SemaphoreType.DMA,
          pltpu.SemaphoreType.DMA,
          pltpu.SemaphoreType.DMA,
          pltpu.SemaphoreType.DMA,
          pltpu.VMEM_SHARED((TBL, D), jnp.float32),
      ],
  )


def _dense_mid_body(a_ref, h_ref, wrelT_ref, wrootT_ref, b_ref, o_ref):
  y = jnp.dot(a_ref[...], wrelT_ref[...], preferred_element_type=jnp.float32)
  y += jnp.dot(h_ref[...], wrootT_ref[...], preferred_element_type=jnp.float32)
  y += b_ref[...]
  o_ref[...] = jnp.maximum(y, 0.0)


def _final_body(h_ref, wgT_ref, bg_ref, o_ref):
  o_ref[...] = (
      jnp.dot(h_ref[...], wgT_ref[...], preferred_element_type=jnp.float32)
      + bg_ref[...])


_R = 2000


def _dense_mid(agg, h, wrelT, wrootT, b2d):
  grid = (N_NODES // _R,)
  return pl.pallas_call(
      _dense_mid_body,
      grid=grid,
      in_specs=[
          pl.BlockSpec((_R, D), lambda i: (i, 0)),
          pl.BlockSpec((_R, D), lambda i: (i, 0)),
          pl.BlockSpec((D, D), lambda i: (0, 0)),
          pl.BlockSpec((D, D), lambda i: (0, 0)),
          pl.BlockSpec((1, D), lambda i: (0, 0)),
      ],
      out_specs=pl.BlockSpec((_R, D), lambda i: (i, 0)),
      out_shape=jax.ShapeDtypeStruct((N_NODES, D), jnp.float32),
  )(agg, h, wrelT, wrootT, b2d)


def _final(h, wgT, bg2d):
  grid = (N_NODES // _R,)
  return pl.pallas_call(
      _final_body,
      grid=grid,
      in_specs=[
          pl.BlockSpec((_R, D), lambda i: (i, 0)),
          pl.BlockSpec((D, D), lambda i: (0, 0)),
          pl.BlockSpec((1, D), lambda i: (0, 0)),
      ],
      out_specs=pl.BlockSpec((_R, D), lambda i: (i, 0)),
      out_shape=jax.ShapeDtypeStruct((N_NODES, D), jnp.float32),
  )(h, wgT, bg2d)


def kernel(x, edge_index, Wrel0, brel0, Wroot0, Wrel1, brel1, Wroot1, Wrel2,
           brel2, Wroot2, Wg, bg):
  n_edges = edge_index.shape[1]
  src = edge_index[0]
  dst = edge_index[1]
  pkp = _make_part()(src, dst)
  agg_fn = _make_agg()

  wrelT = jnp.stack([Wrel0.T, Wrel1.T, Wrel2.T])
  wrootT = jnp.stack([Wroot0.T, Wroot1.T, Wroot2.T])
  b2 = jnp.stack([brel0.reshape(1, D), brel1.reshape(1, D),
                  brel2.reshape(1, D)])

  def layer(h, ws):
    wrelT_i, wrootT_i, b_i = ws
    agg = agg_fn(h, pkp)
    h2 = _dense_mid(agg, h, wrelT_i, wrootT_i, b_i)
    return h2, None

  h3, _ = lax.scan(layer, x, (wrelT, wrootT, b2))
  return _final(h3, Wg.T, bg.reshape(1, D))


# speedup vs baseline: 1.0890x; 1.0425x over previous
import functools

import jax
import jax.numpy as jnp
from jax import lax
from jax.experimental import pallas as pl
from jax.experimental.pallas import tpu as pltpu
from jax.experimental.pallas import tpu_sc as plsc

N_NODES = 10000
D = 128
NC = 2
NS = 16
K = 80
HALF = 5120
TBL = 5248
TRASH = HALF
TBL_PER_TILE = TBL // NS
OUT_PER_TILE = HALF // NS
N_PAD = 2 * HALF

NV = 64
EPV = 320000 // NV
CAP = 5200
DUMP = 5088
CNTPOS = 5184
BPT = NV // NS
NGRP = 5120 // 16


def _prefix16(x):
  lanes = lax.iota(jnp.int32, 16)
  p = x
  for sh in (1, 2, 4, 8):
    idx = jnp.maximum(lanes - sh, 0)
    g = p.at[idx].get(mode="promise_in_bounds")
    p = p + jnp.where(lanes >= sh, g, 0)
  return p


def _part_body(src_hbm, dst_hbm, pkp_hbm, src_blk, dst_blk,
               pos_stA, vpk_stA, pos_stB, vpk_stB, psem0, psem1, bpk_sh):
  pos_sts = (pos_stA, pos_stB)
  vpk_sts = (vpk_stA, vpk_stB)
  psems = (psem0, psem1)
  c = lax.axis_index("c")
  s = lax.axis_index("s")
  half_base = c * HALF
  lanes = lax.iota(jnp.int32, 16)
  neg16 = jnp.full((16,), -1, jnp.int32)
  trash_pk = lanes + jnp.int32(TRASH << 16)

  for j in range(BPT):
    v = BPT * s + j
    abs_base = v * CAP
    pltpu.sync_copy(src_hbm.at[pl.ds(v * EPV, EPV)],
                    src_blk.at[pl.ds(0, EPV)])
    pltpu.sync_copy(dst_hbm.at[pl.ds(v * EPV, EPV)],
                    dst_blk.at[pl.ds(0, EPV)])
    d = dst_blk[pl.ds(EPV - 8, 16)]
    dst_blk[pl.ds(EPV - 8, 16)] = jnp.where(lanes < 8, d, neg16)
    for q in range(EPV + 8, NGRP * 16, 16):
      dst_blk[pl.ds(q, 16)] = neg16

    def pair(pi, cnt):
      for sb in range(2):
        bi = 2 * pi + sb

        @pl.when(pi >= 1)
        def _():
          pltpu.make_async_copy(vpk_sts[sb], bpk_sh.at[pos_sts[sb]],
                                psems[sb]).wait()

        for gi in range(8):
          s16 = src_blk[pl.ds(bi * 128 + gi * 16, 16)]
          d16 = dst_blk[pl.ds(bi * 128 + gi * 16, 16)]
          loc = d16 - half_base
          ok = (loc >= 0) & (loc < HALF)
          ok_i = jnp.where(ok, jnp.int32(1), jnp.int32(0))
          pref = _prefix16(ok_i)
          pos = jnp.where(ok, abs_base + cnt + pref - 1,
                          abs_base + DUMP + lanes)
          pos_sts[sb][pl.ds(gi * 16, 16)] = pos
          vpk_sts[sb][pl.ds(gi * 16, 16)] = s16 | (loc << 16)
          cnt = cnt + pref[15]
        pltpu.async_copy(vpk_sts[sb], bpk_sh.at[pos_sts[sb]], psems[sb])
      return cnt

    cnt = lax.fori_loop(0, NGRP // 16, pair, jnp.int32(0))
    for sb in range(2):
      pltpu.make_async_copy(vpk_sts[sb], bpk_sh.at[pos_sts[sb]],
                            psems[sb]).wait()

    nch = (cnt + (K - 1)) // K
    for gi in range(8):
      if gi < 5:
        pos_stA[pl.ds(gi * 16, 16)] = abs_base + cnt + gi * 16 + lanes
        vpk_stA[pl.ds(gi * 16, 16)] = trash_pk
      elif gi == 5:
        pos_stA[pl.ds(gi * 16, 16)] = abs_base + CNTPOS + lanes
        vpk_stA[pl.ds(gi * 16, 16)] = jnp.zeros((16,), jnp.int32) + nch
      else:
        pos_stA[pl.ds(gi * 16, 16)] = abs_base + DUMP + lanes
        vpk_stA[pl.ds(gi * 16, 16)] = trash_pk
    pltpu.sync_copy(vpk_stA, bpk_sh.at[pos_stA])

  for j in range(BPT):
    v = BPT * s + j
    pltpu.sync_copy(bpk_sh.at[pl.ds(v * CAP, CAP)], src_blk)
    pltpu.sync_copy(src_blk, pkp_hbm.at[c, v])


def _make_part():
  mesh = plsc.VectorSubcoreMesh(core_axis_name="c", subcore_axis_name="s",
                                num_cores=NC)
  return pl.kernel(
      _part_body,
      out_type=jax.ShapeDtypeStruct((NC, NV, CAP), jnp.int32),
      mesh=mesh,
      scratch_types=[
          pltpu.VMEM((CAP,), jnp.int32),
          pltpu.VMEM((CAP,), jnp.int32),
          pltpu.VMEM((128,), jnp.int32),
          pltpu.VMEM((128,), jnp.int32),
          pltpu.VMEM((128,), jnp.int32),
          pltpu.VMEM((128,), jnp.int32),
          pltpu.SemaphoreType.DMA,
          pltpu.SemaphoreType.DMA,
          pltpu.VMEM_SHARED((NV * CAP,), jnp.int32),
      ],
  )


def _agg_body(h_hbm, pkp_hbm, out_hbm, pk_blk,
              srcc0, srcc1, srcc2, srcc3, srcc4,
              dstc0, dstc1, dstc2, dstc3, dstc4,
              rows0, rows1, rows2, rows3, rows4,
              gsem0, gsem1, gsem2, gsem3, gsem4,
              ssem0, ssem1, ssem2, ssem3, ssem4,
              agg_sh):
  c = lax.axis_index("c")
  s = lax.axis_index("s")
  half_base = c * HALF
  srcc = (srcc0, srcc1, srcc2, srcc3, srcc4)
  dstc = (dstc0, dstc1, dstc2, dstc3, dstc4)
  rows = (rows0, rows1, rows2, rows3, rows4)
  gsem = (gsem0, gsem1, gsem2, gsem3, gsem4)
  ssem = (ssem0, ssem1, ssem2, ssem3, ssem4)

  zeros16 = jnp.zeros((16,), jnp.float32)

  def zrow(i, _):
    for j in range(8):
      rows0[i, pl.ds(j * 16, 16)] = zeros16
    return 0

  lax.fori_loop(0, K, zrow, 0)
  for q in range(4):
    pltpu.sync_copy(rows0, agg_sh.at[pl.ds(s * TBL_PER_TILE + q * K, K)])
  pltpu.sync_copy(rows0.at[pl.ds(0, TBL_PER_TILE - 4 * K)],
                  agg_sh.at[pl.ds(s * TBL_PER_TILE + 4 * K,
                                  TBL_PER_TILE - 4 * K)])
  plsc.subcore_barrier()

  def stage(g, b):
    for j in range(K // 16):
      e = pk_blk[pl.ds(g * K + j * 16, 16)]
      srcc[b][pl.ds(j * 16, 16)] = e & 0xFFFF
      dstc[b][pl.ds(j * 16, 16)] = e >> 16

  def start_gather(g, b):
    pltpu.async_copy(h_hbm.at[srcc[b]], rows[b], gsem[b])

  def wait_gather(b):
    pltpu.make_async_copy(h_hbm.at[srcc[b]], rows[b], gsem[b]).wait()

  def start_scatter(b):
    pltpu.async_copy(rows[b], agg_sh.at[dstc[b]], ssem[b], add=True)

  def wait_scatter(b):
    pltpu.make_async_copy(rows[b], agg_sh.at[dstc[b]], ssem[b]).wait()

  for j in range(BPT):
    v = BPT * s + j
    pltpu.sync_copy(pkp_hbm.at[c, v], pk_blk)
    n_ch = pk_blk[pl.ds(CNTPOS, 16)][0]

    for b in range(3):

      @pl.when(b < n_ch)
      def _():
        stage(b, b)
        start_gather(b, b)

    def quint(qq, _):
      for b in range(5):
        g = 5 * qq + b

        @pl.when(g < n_ch)
        def _():
          wait_gather(b)
          start_scatter(b)
          bb = (b + 3) % 5

          @pl.when(g >= 2)
          def _():
            wait_scatter(bb)

          @pl.when(g + 3 < n_ch)
          def _():
            stage(g + 3, bb)
            start_gather(g + 3, bb)
      return 0

    lax.fori_loop(0, (n_ch + 4) // 5, quint, 0)

    for b in range(5):

      @pl.when((n_ch >= 2) & ((n_ch - 2) % 5 == b))
      def _():
        wait_scatter(b)
    for b in range(5):

      @pl.when((n_ch >= 1) & ((n_ch - 1) % 5 == b))
      def _():
        wait_scatter(b)

  plsc.subcore_barrier()

  for q in range(4):
    pltpu.sync_copy(agg_sh.at[pl.ds(s * OUT_PER_TILE + q * K, K)], rows[q])
    pltpu.sync_copy(
        rows[q], out_hbm.at[pl.ds(half_base + s * OUT_PER_TILE + q * K, K)])


def _make_agg():
  mesh = plsc.VectorSubcoreMesh(core_axis_name="c", subcore_axis_name="s",
                                num_cores=NC)
  return pl.kernel(
      _agg_body,
      out_type=jax.ShapeDtypeStruct((N_PAD, D), jnp.float32),
      mesh=mesh,
      scratch_types=[
          pltpu.VMEM((CAP,), jnp.int32),
          pltpu.VMEM((K,), jnp.int32),
          pltpu.VMEM((K,), jnp.int32),
          pltpu.VMEM((K,), jnp.int32),
          pltpu.VMEM((K,), jnp.int32),
          pltpu.VMEM((K,), jnp.int32),
          pltpu.VMEM((K,), jnp.int32),
          pltpu.VMEM((K,), jnp.int32),
          pltpu.VMEM((K,), jnp.int32),
          pltpu.VMEM((K,), jnp.int32),
          pltpu.VMEM((K,), jnp.int32),
          pltpu.VMEM((K, D), jnp.float32),
          pltpu.VMEM((K, D), jnp.float32),
          pltpu.VMEM((K, D), jnp.float32),
          pltpu.VMEM((K, D), jnp.float32),
          pltpu.VMEM((K, D), jnp.float32),
          pltpu.SemaphoreType.DMA,
          pltpu.SemaphoreType.DMA,
          pltpu.SemaphoreType.DMA,
          pltpu.SemaphoreType.DMA,
          pltpu.SemaphoreType.DMA,
          pltpu.SemaphoreType.DMA,
          pltpu.SemaphoreType.DMA,
          pltpu.SemaphoreType.DMA,
          pltpu.SemaphoreType.DMA,
          pltpu.SemaphoreType.DMA,
          pltpu.VMEM_SHARED((TBL, D), jnp.float32),
      ],
  )


def _dense_mid_body(a_ref, h_ref, wrelT_ref, wrootT_ref, b_ref, o_ref):
  y = jnp.dot(a_ref[...], wrelT_ref[...], preferred_element_type=jnp.float32)
  y += jnp.dot(h_ref[...], wrootT_ref[...], preferred_element_type=jnp.float32)
  y += b_ref[...]
  o_ref[...] = jnp.maximum(y, 0.0)


def _final_body(h_ref, wgT_ref, bg_ref, o_ref):
  o_ref[...] = (
      jnp.dot(h_ref[...], wgT_ref[...], preferred_element_type=jnp.float32)
      + bg_ref[...])


_R = 2000


def _dense_mid(agg, h, wrelT, wrootT, b2d):
  grid = (N_NODES // _R,)
  return pl.pallas_call(
      _dense_mid_body,
      grid=grid,
      in_specs=[
          pl.BlockSpec((_R, D), lambda i: (i, 0)),
          pl.BlockSpec((_R, D), lambda i: (i, 0)),
          pl.BlockSpec((D, D), lambda i: (0, 0)),
          pl.BlockSpec((D, D), lambda i: (0, 0)),
          pl.BlockSpec((1, D), lambda i: (0, 0)),
      ],
      out_specs=pl.BlockSpec((_R, D), lambda i: (i, 0)),
      out_shape=jax.ShapeDtypeStruct((N_NODES, D), jnp.float32),
  )(agg, h, wrelT, wrootT, b2d)


def _final(h, wgT, bg2d):
  grid = (N_NODES // _R,)
  return pl.pallas_call(
      _final_body,
      grid=grid,
      in_specs=[
          pl.BlockSpec((_R, D), lambda i: (i, 0)),
          pl.BlockSpec((D, D), lambda i: (0, 0)),
          pl.BlockSpec((1, D), lambda i: (0, 0)),
      ],
      out_specs=pl.BlockSpec((_R, D), lambda i: (i, 0)),
      out_shape=jax.ShapeDtypeStruct((N_NODES, D), jnp.float32),
  )(h, wgT, bg2d)


def kernel(x, edge_index, Wrel0, brel0, Wroot0, Wrel1, brel1, Wroot1, Wrel2,
           brel2, Wroot2, Wg, bg):
  n_edges = edge_index.shape[1]
  src = edge_index[0]
  dst = edge_index[1]
  pkp = _make_part()(src, dst)
  agg_fn = _make_agg()

  wrelT = jnp.stack([Wrel0.T, Wrel1.T, Wrel2.T])
  wrootT = jnp.stack([Wroot0.T, Wroot1.T, Wroot2.T])
  b2 = jnp.stack([brel0.reshape(1, D), brel1.reshape(1, D),
                  brel2.reshape(1, D)])

  def layer(h, ws):
    wrelT_i, wrootT_i, b_i = ws
    agg = agg_fn(h, pkp)
    h2 = _dense_mid(agg, h, wrelT_i, wrootT_i, b_i)
    return h2, None

  h3, _ = lax.scan(layer, x, (wrelT, wrootT, b2))
  return _final(h3, Wg.T, bg.reshape(1, D))


# speedup vs baseline: 1.1151x; 1.0240x over previous
import functools

import jax
import jax.numpy as jnp
from jax import lax
from jax.experimental import pallas as pl
from jax.experimental.pallas import tpu as pltpu
from jax.experimental.pallas import tpu_sc as plsc

N_NODES = 10000
D = 128
NC = 2
NS = 16
K = 80
HALF = 5120
TBL = 5248
TRASH = HALF
TBL_PER_TILE = TBL // NS
OUT_PER_TILE = HALF // NS
N_PAD = 2 * HALF

NV = 64
EPV = 320000 // NV
CAP = 5200
DUMP = 5088
CNTPOS = 5184
BPT = NV // NS
NGRP = 5120 // 16


def _prefix16(x):
  lanes = lax.iota(jnp.int32, 16)
  p = x
  for sh in (1, 2, 4, 8):
    idx = jnp.maximum(lanes - sh, 0)
    g = p.at[idx].get(mode="promise_in_bounds")
    p = p + jnp.where(lanes >= sh, g, 0)
  return p


def _part_body(src_hbm, dst_hbm, pkp_hbm, src_blk, dst_blk,
               pos_stA, vpk_stA, pos_stB, vpk_stB, psem0, psem1, bpk_sh):
  pos_sts = (pos_stA, pos_stB)
  vpk_sts = (vpk_stA, vpk_stB)
  psems = (psem0, psem1)
  c = lax.axis_index("c")
  s = lax.axis_index("s")
  half_base = c * HALF
  lanes = lax.iota(jnp.int32, 16)
  neg16 = jnp.full((16,), -1, jnp.int32)
  trash_pk = lanes + jnp.int32(TRASH << 16)

  for j in range(BPT):
    v = BPT * s + j
    abs_base = v * CAP
    pltpu.sync_copy(src_hbm.at[pl.ds(v * EPV, EPV)],
                    src_blk.at[pl.ds(0, EPV)])
    pltpu.sync_copy(dst_hbm.at[pl.ds(v * EPV, EPV)],
                    dst_blk.at[pl.ds(0, EPV)])
    d = dst_blk[pl.ds(EPV - 8, 16)]
    dst_blk[pl.ds(EPV - 8, 16)] = jnp.where(lanes < 8, d, neg16)
    for q in range(EPV + 8, NGRP * 16, 16):
      dst_blk[pl.ds(q, 16)] = neg16

    def pair(pi, cnt):
      for sb in range(2):
        bi = 2 * pi + sb

        @pl.when(pi >= 1)
        def _():
          pltpu.make_async_copy(vpk_sts[sb], bpk_sh.at[pos_sts[sb]],
                                psems[sb]).wait()

        for gi in range(8):
          s16 = src_blk[pl.ds(bi * 128 + gi * 16, 16)]
          d16 = dst_blk[pl.ds(bi * 128 + gi * 16, 16)]
          loc = d16 - half_base
          ok = (loc >= 0) & (loc < HALF)
          ok_i = jnp.where(ok, jnp.int32(1), jnp.int32(0))
          pref = _prefix16(ok_i)
          pos = jnp.where(ok, abs_base + cnt + pref - 1,
                          abs_base + DUMP + lanes)
          pos_sts[sb][pl.ds(gi * 16, 16)] = pos
          vpk_sts[sb][pl.ds(gi * 16, 16)] = s16 | (loc << 16)
          cnt = cnt + pref[15]
        pltpu.async_copy(vpk_sts[sb], bpk_sh.at[pos_sts[sb]], psems[sb])
      return cnt

    cnt = lax.fori_loop(0, NGRP // 16, pair, jnp.int32(0))
    for sb in range(2):
      pltpu.make_async_copy(vpk_sts[sb], bpk_sh.at[pos_sts[sb]],
                            psems[sb]).wait()

    nch = (cnt + (K - 1)) // K
    for gi in range(8):
      if gi < 5:
        pos_stA[pl.ds(gi * 16, 16)] = abs_base + cnt + gi * 16 + lanes
        vpk_stA[pl.ds(gi * 16, 16)] = trash_pk
      elif gi == 5:
        pos_stA[pl.ds(gi * 16, 16)] = abs_base + CNTPOS + lanes
        vpk_stA[pl.ds(gi * 16, 16)] = jnp.zeros((16,), jnp.int32) + nch
      else:
        pos_stA[pl.ds(gi * 16, 16)] = abs_base + DUMP + lanes
        vpk_stA[pl.ds(gi * 16, 16)] = trash_pk
    pltpu.sync_copy(vpk_stA, bpk_sh.at[pos_stA])

  for j in range(BPT):
    v = BPT * s + j
    pltpu.sync_copy(bpk_sh.at[pl.ds(v * CAP, CAP)], src_blk)
    pltpu.sync_copy(src_blk, pkp_hbm.at[c, v])


def _make_part():
  mesh = plsc.VectorSubcoreMesh(core_axis_name="c", subcore_axis_name="s",
                                num_cores=NC)
  return pl.kernel(
      _part_body,
      out_type=jax.ShapeDtypeStruct((NC, NV, CAP), jnp.int32),
      mesh=mesh,
      scratch_types=[
          pltpu.VMEM((CAP,), jnp.int32),
          pltpu.VMEM((CAP,), jnp.int32),
          pltpu.VMEM((128,), jnp.int32),
          pltpu.VMEM((128,), jnp.int32),
          pltpu.VMEM((128,), jnp.int32),
          pltpu.VMEM((128,), jnp.int32),
          pltpu.SemaphoreType.DMA,
          pltpu.SemaphoreType.DMA,
          pltpu.VMEM_SHARED((NV * CAP,), jnp.int32),
      ],
  )


def _agg_body(h_hbm, pkp_hbm, out_hbm, pk_blk,
              srcc0, srcc1, srcc2, srcc3, srcc4,
              dstc0, dstc1, dstc2, dstc3, dstc4,
              rows0, rows1, rows2, rows3, rows4,
              gsem0, gsem1, gsem2, gsem3, gsem4,
              ssem0, ssem1, ssem2, ssem3, ssem4,
              agg_sh):
  c = lax.axis_index("c")
  s = lax.axis_index("s")
  half_base = c * HALF
  srcc = (srcc0, srcc1, srcc2, srcc3, srcc4)
  dstc = (dstc0, dstc1, dstc2, dstc3, dstc4)
  rows = (rows0, rows1, rows2, rows3, rows4)
  gsem = (gsem0, gsem1, gsem2, gsem3, gsem4)
  ssem = (ssem0, ssem1, ssem2, ssem3, ssem4)

  zeros16 = jnp.zeros((16,), jnp.float32)

  def zrow(i, _):
    for j in range(8):
      rows0[i, pl.ds(j * 16, 16)] = zeros16
    return 0

  lax.fori_loop(0, K, zrow, 0)
  for q in range(4):
    pltpu.sync_copy(rows0, agg_sh.at[pl.ds(s * TBL_PER_TILE + q * K, K)])
  pltpu.sync_copy(rows0.at[pl.ds(0, TBL_PER_TILE - 4 * K)],
                  agg_sh.at[pl.ds(s * TBL_PER_TILE + 4 * K,
                                  TBL_PER_TILE - 4 * K)])
  plsc.subcore_barrier()

  def stage(g, b):
    for j in range(K // 16):
      e = pk_blk[pl.ds(g * K + j * 16, 16)]
      srcc[b][pl.ds(j * 16, 16)] = e & 0xFFFF
      dstc[b][pl.ds(j * 16, 16)] = e >> 16

  def start_gather(g, b):
    pltpu.async_copy(h_hbm.at[srcc[b]], rows[b], gsem[b])

  def wait_gather(b):
    pltpu.make_async_copy(h_hbm.at[srcc[b]], rows[b], gsem[b]).wait()

  def start_scatter(b):
    pltpu.async_copy(rows[b], agg_sh.at[dstc[b]], ssem[b], add=True)

  def wait_scatter(b):
    pltpu.make_async_copy(rows[b], agg_sh.at[dstc[b]], ssem[b]).wait()

  for j in range(BPT):
    v = BPT * s + j
    pltpu.sync_copy(pkp_hbm.at[c, v], pk_blk)
    n_ch = pk_blk[pl.ds(CNTPOS, 16)][0]

    for b in range(4):

      @pl.when(b < n_ch)
      def _():
        stage(b, b)
        start_gather(b, b)

    def quint(qq, _):
      for b in range(5):
        g = 5 * qq + b

        @pl.when(g < n_ch)
        def _():
          wait_gather(b)
          start_scatter(b)
          bb = (b + 4) % 5

          @pl.when(g >= 1)
          def _():
            wait_scatter(bb)

          @pl.when(g + 4 < n_ch)
          def _():
            stage(g + 4, bb)
            start_gather(g + 4, bb)
      return 0

    lax.fori_loop(0, (n_ch + 4) // 5, quint, 0)

    for b in range(5):

      @pl.when((n_ch >= 1) & ((n_ch - 1) % 5 == b))
      def _():
        wait_scatter(b)

  plsc.subcore_barrier()

  for q in range(4):
    pltpu.sync_copy(agg_sh.at[pl.ds(s * OUT_PER_TILE + q * K, K)], rows[q])
    pltpu.sync_copy(
        rows[q], out_hbm.at[pl.ds(half_base + s * OUT_PER_TILE + q * K, K)])


def _make_agg():
  mesh = plsc.VectorSubcoreMesh(core_axis_name="c", subcore_axis_name="s",
                                num_cores=NC)
  return pl.kernel(
      _agg_body,
      out_type=jax.ShapeDtypeStruct((N_PAD, D), jnp.float32),
      mesh=mesh,
      scratch_types=[
          pltpu.VMEM((CAP,), jnp.int32),
          pltpu.VMEM((K,), jnp.int32),
          pltpu.VMEM((K,), jnp.int32),
          pltpu.VMEM((K,), jnp.int32),
          pltpu.VMEM((K,), jnp.int32),
          pltpu.VMEM((K,), jnp.int32),
          pltpu.VMEM((K,), jnp.int32),
          pltpu.VMEM((K,), jnp.int32),
          pltpu.VMEM((K,), jnp.int32),
          pltpu.VMEM((K,), jnp.int32),
          pltpu.VMEM((K,), jnp.int32),
          pltpu.VMEM((K, D), jnp.float32),
          pltpu.VMEM((K, D), jnp.float32),
          pltpu.VMEM((K, D), jnp.float32),
          pltpu.VMEM((K, D), jnp.float32),
          pltpu.VMEM((K, D), jnp.float32),
          pltpu.SemaphoreType.DMA,
          pltpu.SemaphoreType.DMA,
          pltpu.SemaphoreType.DMA,
          pltpu.SemaphoreType.DMA,
          pltpu.SemaphoreType.DMA,
          pltpu.SemaphoreType.DMA,
          pltpu.SemaphoreType.DMA,
          pltpu.SemaphoreType.DMA,
          pltpu.SemaphoreType.DMA,
          pltpu.SemaphoreType.DMA,
          pltpu.VMEM_SHARED((TBL, D), jnp.float32),
      ],
  )


def _dense_mid_body(a_ref, h_ref, wrelT_ref, wrootT_ref, b_ref, o_ref):
  y = jnp.dot(a_ref[...], wrelT_ref[...], preferred_element_type=jnp.float32)
  y += jnp.dot(h_ref[...], wrootT_ref[...], preferred_element_type=jnp.float32)
  y += b_ref[...]
  o_ref[...] = jnp.maximum(y, 0.0)


def _final_body(h_ref, wgT_ref, bg_ref, o_ref):
  o_ref[...] = (
      jnp.dot(h_ref[...], wgT_ref[...], preferred_element_type=jnp.float32)
      + bg_ref[...])


_R = 2000


def _dense_mid(agg, h, wrelT, wrootT, b2d):
  grid = (N_NODES // _R,)
  return pl.pallas_call(
      _dense_mid_body,
      grid=grid,
      in_specs=[
          pl.BlockSpec((_R, D), lambda i: (i, 0)),
          pl.BlockSpec((_R, D), lambda i: (i, 0)),
          pl.BlockSpec((D, D), lambda i: (0, 0)),
          pl.BlockSpec((D, D), lambda i: (0, 0)),
          pl.BlockSpec((1, D), lambda i: (0, 0)),
      ],
      out_specs=pl.BlockSpec((_R, D), lambda i: (i, 0)),
      out_shape=jax.ShapeDtypeStruct((N_NODES, D), jnp.float32),
  )(agg, h, wrelT, wrootT, b2d)


def _final(h, wgT, bg2d):
  grid = (N_NODES // _R,)
  return pl.pallas_call(
      _final_body,
      grid=grid,
      in_specs=[
          pl.BlockSpec((_R, D), lambda i: (i, 0)),
          pl.BlockSpec((D, D), lambda i: (0, 0)),
          pl.BlockSpec((1, D), lambda i: (0, 0)),
      ],
      out_specs=pl.BlockSpec((_R, D), lambda i: (i, 0)),
      out_shape=jax.ShapeDtypeStruct((N_NODES, D), jnp.float32),
  )(h, wgT, bg2d)


def kernel(x, edge_index, Wrel0, brel0, Wroot0, Wrel1, brel1, Wroot1, Wrel2,
           brel2, Wroot2, Wg, bg):
  n_edges = edge_index.shape[1]
  src = edge_index[0]
  dst = edge_index[1]
  pkp = _make_part()(src, dst)
  agg_fn = _make_agg()

  wrelT = jnp.stack([Wrel0.T, Wrel1.T, Wrel2.T])
  wrootT = jnp.stack([Wroot0.T, Wroot1.T, Wroot2.T])
  b2 = jnp.stack([brel0.reshape(1, D), brel1.reshape(1, D),
                  brel2.reshape(1, D)])

  def layer(h, ws):
    wrelT_i, wrootT_i, b_i = ws
    agg = agg_fn(h, pkp)
    h2 = _dense_mid(agg, h, wrelT_i, wrootT_i, b_i)
    return h2, None

  h3, _ = lax.scan(layer, x, (wrelT, wrootT, b2))
  return _final(h3, Wg.T, bg.reshape(1, D))
